# baseline (speedup 1.0000x reference)
import functools

import jax
import jax.numpy as jnp
from jax import lax
from jax.experimental import pallas as pl
from jax.experimental.pallas import tpu as pltpu
from jax.experimental.pallas import tpu_sc as plsc

N = 10000
E = 320000
D = 128
TEM = 0.7
DROP = 0.01

NS = 16
CH = 96
CPT = 212
EPT = CH * CPT
EPAD = EPT * NS
SENT = N
ACC_R = 10016
RPT = 632
ORPT = 624
OREM = N - ORPT * NS
DCH = 3
DPT = DCH * CH
DCAP = DPT * NS
NSL = 4

_mesh = plsc.VectorSubcoreMesh(core_axis_name="c", subcore_axis_name="s")


@functools.partial(
    pl.kernel,
    out_type=[jax.ShapeDtypeStruct((N, D), jnp.float32)] * 4,
    mesh=_mesh,
    scratch_types=[
        [pltpu.VMEM((CH,), jnp.int32) for _ in range(NSL)],
        [pltpu.VMEM((CH,), jnp.int32) for _ in range(NSL)],
        [pltpu.VMEM((CH, D), jnp.float32) for _ in range(NSL)],
        pltpu.VMEM_SHARED((ACC_R, D), jnp.float32),
        [pltpu.SemaphoreType.DMA for _ in range(NSL)],
        [pltpu.SemaphoreType.DMA for _ in range(NSL)],
        [pltpu.SemaphoreType.DMA for _ in range(NSL)],
    ],
)
def _sc_msgpass(feat_u, feat_i, s_ub, d_ub, s_bu, d_bu,
                ds_ub, dd_ub, ds_bu, dd_bu,
                hpos_ub, hpos_bu, dump_ub, dump_bu,
                sidx, didx, stage, acc, ssem, dsem, gsem):
    c = lax.axis_index("c")
    s = lax.axis_index("s")

    def run(feat, src, dst, dsrc, ddst, hpos, dump):
        base = s * EPT

        def istart(j, b):
            pltpu.async_copy(src.at[pl.ds(base + j * CH, CH)], sidx[b],
                             ssem[b])
            pltpu.async_copy(dst.at[pl.ds(base + j * CH, CH)], didx[b],
                             dsem[b])

        def iwait(b):
            pltpu.make_async_copy(src.at[pl.ds(0, CH)], sidx[b],
                                  ssem[b]).wait()
            pltpu.make_async_copy(dst.at[pl.ds(0, CH)], didx[b],
                                  dsem[b]).wait()

        def gstart(b):
            pltpu.async_copy(feat.at[sidx[b]], stage[b], gsem[b])

        def gwait(b):
            pltpu.make_async_copy(feat.at[sidx[b]], stage[b], gsem[b]).wait()

        def zrow(i, _):
            for g in range(D // 16):
                stage[0][i, pl.ds(g * 16, 16)] = jnp.zeros((16,), jnp.float32)
            return 0
        lax.fori_loop(0, CH, zrow, 0)
        for t in range(5):
            pltpu.sync_copy(stage[0], acc.at[pl.ds(s * RPT + t * CH, CH)])

        @pl.when(s < NS - 1)
        def _():
            pltpu.sync_copy(stage[0], acc.at[pl.ds(s * RPT + 480, CH)])
            pltpu.sync_copy(stage[0].at[pl.ds(0, 56)],
                            acc.at[pl.ds(s * RPT + 576, 56)])

        @pl.when(s == NS - 1)
        def _():
            pltpu.sync_copy(stage[0].at[pl.ds(0, 56)],
                            acc.at[pl.ds(s * RPT + 480, 56)])
        plsc.subcore_barrier()

        for k in range(NSL):
            istart(k, k)
        iwait(0)
        gstart(0)
        iwait(1)
        gstart(1)
        iwait(2)
        gstart(2)

        def body(k, _):
            for b in range(NSL):
                j = k * NSL + b
                nb = (b + 3) % NSL

                @pl.when(j + 3 < CPT)
                def _():
                    iwait(nb)
                    gstart(nb)

                gwait(b)
                pltpu.sync_copy(stage[b], acc.at[didx[b]], add=True)

                @pl.when(j + NSL < CPT)
                def _():
                    istart(j + NSL, b)
            return 0
        lax.fori_loop(0, CPT // NSL, body, 0)
        plsc.subcore_barrier()

        pltpu.sync_copy(acc.at[pl.ds(s * ORPT, ORPT)],
                        hpos.at[pl.ds(s * ORPT, ORPT)])

        @pl.when(s == NS - 1)
        def _():
            pltpu.sync_copy(acc.at[pl.ds(ORPT * NS, OREM)],
                            hpos.at[pl.ds(ORPT * NS, OREM)])
        plsc.subcore_barrier()

        for ic in range(DCH):
            off = s * DPT + ic * CH
            pltpu.sync_copy(dsrc.at[pl.ds(off, CH)], sidx[0])
            pltpu.sync_copy(ddst.at[pl.ds(off, CH)], didx[0])
            pltpu.async_copy(feat.at[sidx[0]], stage[0], gsem[0]).wait()
            pltpu.sync_copy(stage[0], acc.at[didx[0]], add=True)
        plsc.subcore_barrier()

        pltpu.sync_copy(acc.at[pl.ds(s * ORPT, ORPT)],
                        dump.at[pl.ds(s * ORPT, ORPT)])

        @pl.when(s == NS - 1)
        def _():
            pltpu.sync_copy(acc.at[pl.ds(ORPT * NS, OREM)],
                            dump.at[pl.ds(ORPT * NS, OREM)])

    @pl.when(c == 0)
    def _():
        run(feat_u, s_ub, d_ub, ds_ub, dd_ub, hpos_ub, dump_ub)

    @pl.when(c == 1)
    def _():
        run(feat_i, s_bu, d_bu, ds_bu, dd_bu, hpos_bu, dump_bu)


def _loss_body(aub, dub, abu, dbu, out):
    def one(a, dmp):
        b = 2.0 * a - dmp
        num = jnp.sum(a * b, axis=1)
        na = jnp.sqrt(jnp.sum(a * a, axis=1))
        nb = jnp.sqrt(jnp.sum(b * b, axis=1))
        cos = num / (jnp.maximum(na, 1e-8) * jnp.maximum(nb, 1e-8))
        return jnp.log(jnp.sum(jnp.exp(cos / TEM)))

    out[0, 0] = one(aub[...], dub[...]) + one(abu[...], dbu[...])


_tc_loss = pl.pallas_call(
    _loss_body,
    out_shape=jax.ShapeDtypeStruct((1, 1), jnp.float32),
    out_specs=pl.BlockSpec(memory_space=pltpu.SMEM),
)


def kernel(feat_items, feat_users, edges_ub_src, edges_ub_dst,
           edges_bu_src, edges_bu_dst):
    i32 = jnp.int32
    su = edges_ub_src.astype(i32)
    du = edges_ub_dst.astype(i32)
    sb = edges_bu_src.astype(i32)
    db = edges_bu_dst.astype(i32)

    pad = EPAD - E
    zpad = jnp.zeros((pad,), i32)
    spad = jnp.full((pad,), SENT, i32)
    su_p = jnp.concatenate([su, zpad])
    du_p = jnp.concatenate([du, spad])
    sb_p = jnp.concatenate([sb, zpad])
    db_p = jnp.concatenate([db, spad])

    drop_key = jax.random.key(42)
    k_ub, k_bu = jax.random.split(drop_key)
    keep_ub = jax.random.bernoulli(k_ub, p=1.0 - DROP, shape=(E,))
    keep_bu = jax.random.bernoulli(k_bu, p=1.0 - DROP, shape=(E,))
    ids_ub = jnp.where(~keep_ub, size=DCAP, fill_value=E)[0].astype(i32)
    ids_bu = jnp.where(~keep_bu, size=DCAP, fill_value=E)[0].astype(i32)
    ds_ub = jnp.take(su_p, ids_ub)
    dd_ub = jnp.take(du_p, ids_ub)
    ds_bu = jnp.take(sb_p, ids_bu)
    dd_bu = jnp.take(db_p, ids_bu)

    hpos_ub, hpos_bu, dump_ub, dump_bu = _sc_msgpass(
        feat_users, feat_items, su_p, du_p, sb_p, db_p,
        ds_ub, dd_ub, ds_bu, dd_bu)

    loss = _tc_loss(hpos_ub, dump_ub, hpos_bu, dump_bu)[0, 0]
    return hpos_ub, hpos_bu, loss


# speedup vs baseline: 1.0553x; 1.0553x over previous
import functools

import jax
import jax.numpy as jnp
from jax import lax
from jax.experimental import pallas as pl
from jax.experimental.pallas import tpu as pltpu
from jax.experimental.pallas import tpu_sc as plsc

N = 10000
E = 320000
D = 128
TEM = 0.7
DROP = 0.01

NS = 16
CH = 128
CPT = 159
EPT = CH * CPT
EPAD = EPT * NS
SENT = N
ACC_R = 10016
RPT = 632
ORPT = 624
OREM = N - ORPT * NS
DCH = 2
DPT = DCH * CH
DCAP = DPT * NS
NSL = 3

_mesh = plsc.VectorSubcoreMesh(core_axis_name="c", subcore_axis_name="s")


@functools.partial(
    pl.kernel,
    out_type=[jax.ShapeDtypeStruct((N, D), jnp.float32)] * 4,
    mesh=_mesh,
    scratch_types=[
        [pltpu.VMEM((CH,), jnp.int32) for _ in range(NSL)],
        [pltpu.VMEM((CH,), jnp.int32) for _ in range(NSL)],
        [pltpu.VMEM((CH, D), jnp.float32) for _ in range(NSL)],
        pltpu.VMEM_SHARED((ACC_R, D), jnp.float32),
        [pltpu.SemaphoreType.DMA for _ in range(NSL)],
        [pltpu.SemaphoreType.DMA for _ in range(NSL)],
        [pltpu.SemaphoreType.DMA for _ in range(NSL)],
    ],
)
def _sc_msgpass(feat_u, feat_i, s_ub, d_ub, s_bu, d_bu,
                ds_ub, dd_ub, ds_bu, dd_bu,
                hpos_ub, hpos_bu, dump_ub, dump_bu,
                sidx, didx, stage, acc, ssem, dsem, gsem):
    c = lax.axis_index("c")
    s = lax.axis_index("s")

    def run(feat, src, dst, dsrc, ddst, hpos, dump):
        base = s * EPT

        def istart(j, b):
            pltpu.async_copy(src.at[pl.ds(base + j * CH, CH)], sidx[b],
                             ssem[b])
            pltpu.async_copy(dst.at[pl.ds(base + j * CH, CH)], didx[b],
                             dsem[b])

        def iwait(b):
            pltpu.make_async_copy(src.at[pl.ds(0, CH)], sidx[b],
                                  ssem[b]).wait()
            pltpu.make_async_copy(dst.at[pl.ds(0, CH)], didx[b],
                                  dsem[b]).wait()

        def gstart(b):
            pltpu.async_copy(feat.at[sidx[b]], stage[b], gsem[b])

        def gwait(b):
            pltpu.make_async_copy(feat.at[sidx[b]], stage[b], gsem[b]).wait()

        def zrow(i, _):
            for g in range(D // 16):
                stage[0][i, pl.ds(g * 16, 16)] = jnp.zeros((16,), jnp.float32)
            return 0
        lax.fori_loop(0, CH, zrow, 0)
        for t in range(4):
            pltpu.sync_copy(stage[0], acc.at[pl.ds(s * RPT + t * CH, CH)])

        @pl.when(s < NS - 1)
        def _():
            pltpu.sync_copy(stage[0].at[pl.ds(0, 120)],
                            acc.at[pl.ds(s * RPT + 512, 120)])

        @pl.when(s == NS - 1)
        def _():
            pltpu.sync_copy(stage[0].at[pl.ds(0, 24)],
                            acc.at[pl.ds(s * RPT + 512, 24)])
        plsc.subcore_barrier()

        for k in range(NSL):
            istart(k, k)
        iwait(0)
        gstart(0)
        iwait(1)
        gstart(1)

        def body(k, _):
            for b in range(NSL):
                j = k * NSL + b
                nb = (b + 2) % NSL

                @pl.when(j + 2 < CPT)
                def _():
                    iwait(nb)
                    gstart(nb)

                gwait(b)
                pltpu.sync_copy(stage[b], acc.at[didx[b]], add=True)

                @pl.when(j + NSL < CPT)
                def _():
                    istart(j + NSL, b)
            return 0
        lax.fori_loop(0, CPT // NSL, body, 0)
        plsc.subcore_barrier()

        pltpu.sync_copy(acc.at[pl.ds(s * ORPT, ORPT)],
                        hpos.at[pl.ds(s * ORPT, ORPT)])

        @pl.when(s == NS - 1)
        def _():
            pltpu.sync_copy(acc.at[pl.ds(ORPT * NS, OREM)],
                            hpos.at[pl.ds(ORPT * NS, OREM)])
        plsc.subcore_barrier()

        for ic in range(DCH):
            off = s * DPT + ic * CH
            pltpu.sync_copy(dsrc.at[pl.ds(off, CH)], sidx[0])
            pltpu.sync_copy(ddst.at[pl.ds(off, CH)], didx[0])
            pltpu.async_copy(feat.at[sidx[0]], stage[0], gsem[0]).wait()
            pltpu.sync_copy(stage[0], acc.at[didx[0]], add=True)
        plsc.subcore_barrier()

        pltpu.sync_copy(acc.at[pl.ds(s * ORPT, ORPT)],
                        dump.at[pl.ds(s * ORPT, ORPT)])

        @pl.when(s == NS - 1)
        def _():
            pltpu.sync_copy(acc.at[pl.ds(ORPT * NS, OREM)],
                            dump.at[pl.ds(ORPT * NS, OREM)])

    @pl.when(c == 0)
    def _():
        run(feat_u, s_ub, d_ub, ds_ub, dd_ub, hpos_ub, dump_ub)

    @pl.when(c == 1)
    def _():
        run(feat_i, s_bu, d_bu, ds_bu, dd_bu, hpos_bu, dump_bu)


def _loss_body(aub, dub, abu, dbu, out):
    def one(a, dmp):
        b = 2.0 * a - dmp
        num = jnp.sum(a * b, axis=1)
        na = jnp.sqrt(jnp.sum(a * a, axis=1))
        nb = jnp.sqrt(jnp.sum(b * b, axis=1))
        cos = num / (jnp.maximum(na, 1e-8) * jnp.maximum(nb, 1e-8))
        return jnp.log(jnp.sum(jnp.exp(cos / TEM)))

    out[0, 0] = one(aub[...], dub[...]) + one(abu[...], dbu[...])


_tc_loss = pl.pallas_call(
    _loss_body,
    out_shape=jax.ShapeDtypeStruct((1, 1), jnp.float32),
    out_specs=pl.BlockSpec(memory_space=pltpu.SMEM),
)


def kernel(feat_items, feat_users, edges_ub_src, edges_ub_dst,
           edges_bu_src, edges_bu_dst):
    i32 = jnp.int32
    su = edges_ub_src.astype(i32)
    du = edges_ub_dst.astype(i32)
    sb = edges_bu_src.astype(i32)
    db = edges_bu_dst.astype(i32)

    pad = EPAD - E
    zpad = jnp.zeros((pad,), i32)
    spad = jnp.full((pad,), SENT, i32)
    su_p = jnp.concatenate([su, zpad])
    du_p = jnp.concatenate([du, spad])
    sb_p = jnp.concatenate([sb, zpad])
    db_p = jnp.concatenate([db, spad])

    drop_key = jax.random.key(42)
    k_ub, k_bu = jax.random.split(drop_key)
    keep_ub = jax.random.bernoulli(k_ub, p=1.0 - DROP, shape=(E,))
    keep_bu = jax.random.bernoulli(k_bu, p=1.0 - DROP, shape=(E,))
    ids_ub = jnp.where(~keep_ub, size=DCAP, fill_value=E)[0].astype(i32)
    ids_bu = jnp.where(~keep_bu, size=DCAP, fill_value=E)[0].astype(i32)
    ds_ub = jnp.take(su_p, ids_ub)
    dd_ub = jnp.take(du_p, ids_ub)
    ds_bu = jnp.take(sb_p, ids_bu)
    dd_bu = jnp.take(db_p, ids_bu)

    hpos_ub, hpos_bu, dump_ub, dump_bu = _sc_msgpass(
        feat_users, feat_items, su_p, du_p, sb_p, db_p,
        ds_ub, dd_ub, ds_bu, dd_bu)

    loss = _tc_loss(hpos_ub, dump_ub, hpos_bu, dump_bu)[0, 0]
    return hpos_ub, hpos_bu, loss


# speedup vs baseline: 1.4303x; 1.3553x over previous
import functools

import jax
import jax.numpy as jnp
from jax import lax
from jax.experimental import pallas as pl
from jax.experimental.pallas import tpu as pltpu
from jax.experimental.pallas import tpu_sc as plsc

N = 10000
E = 320000
D = 128
TEM = 0.7
DROP = 0.01

NS = 16
CH = 112
CPT = 180
EPT = CH * CPT
EPAD = EPT * NS
SENT = N
ACC_R = 10016
RPT = 632
ORPT = 624
OREM = N - ORPT * NS
DCH = 3
DLCAP = DCH * CH
NSL = 3

_mesh = plsc.VectorSubcoreMesh(core_axis_name="c", subcore_axis_name="s")


@functools.partial(
    pl.kernel,
    out_type=[jax.ShapeDtypeStruct((N, D), jnp.float32)] * 4,
    mesh=_mesh,
    compiler_params=pltpu.CompilerParams(needs_layout_passes=False),
    scratch_types=[
        [pltpu.VMEM((CH,), jnp.int32) for _ in range(NSL)],
        [pltpu.VMEM((CH,), jnp.int32) for _ in range(NSL)],
        [pltpu.VMEM((CH,), jnp.int32) for _ in range(NSL)],
        [pltpu.VMEM((CH, D), jnp.float32) for _ in range(NSL)],
        pltpu.VMEM((DLCAP + 16,), jnp.int32),
        pltpu.VMEM((DLCAP + 16,), jnp.int32),
        pltpu.VMEM((DCH, CH), jnp.int32),
        pltpu.VMEM_SHARED((ACC_R, D), jnp.float32),
        [pltpu.SemaphoreType.DMA for _ in range(NSL)],
        [pltpu.SemaphoreType.DMA for _ in range(NSL)],
        [pltpu.SemaphoreType.DMA for _ in range(NSL)],
        [pltpu.SemaphoreType.DMA for _ in range(NSL)],
    ],
)
def _sc_msgpass(feat_u, feat_i, s_ub, d_ub, s_bu, d_bu, drop_ub, drop_bu,
                hpos_ub, hpos_bu, dump_ub, dump_bu,
                sidx, didx, kbuf, stage, dls, dld, dld2, acc,
                ssem, dsem, ksem, gsem):
    c = lax.axis_index("c")
    s = lax.axis_index("s")

    def run(feat, src, dst, drop, hpos, dump):
        base = s * EPT

        def istart(j, b):
            pltpu.async_copy(src.at[pl.ds(base + j * CH, CH)], sidx[b],
                             ssem[b])
            pltpu.async_copy(dst.at[pl.ds(base + j * CH, CH)], didx[b],
                             dsem[b])
            pltpu.async_copy(drop.at[pl.ds(base + j * CH, CH)], kbuf[b],
                             ksem[b])

        def iwait(b):
            pltpu.make_async_copy(src.at[pl.ds(0, CH)], sidx[b],
                                  ssem[b]).wait()
            pltpu.make_async_copy(dst.at[pl.ds(0, CH)], didx[b],
                                  dsem[b]).wait()
            pltpu.make_async_copy(drop.at[pl.ds(0, CH)], kbuf[b],
                                  ksem[b]).wait()

        def gstart(b):
            pltpu.async_copy(feat.at[sidx[b]], stage[b], gsem[b])

        def gwait(b):
            pltpu.make_async_copy(feat.at[sidx[b]], stage[b], gsem[b]).wait()

        def zrow(i, _):
            for g in range(D // 16):
                stage[0][i, pl.ds(g * 16, 16)] = jnp.zeros((16,), jnp.float32)
            return 0
        lax.fori_loop(0, CH, zrow, 0)

        def zdl(i, _):
            dls[pl.ds(i * 16, 16)] = jnp.zeros((16,), jnp.int32)
            dld[pl.ds(i * 16, 16)] = jnp.full((16,), SENT, jnp.int32)
            return 0
        lax.fori_loop(0, (DLCAP + 16) // 16, zdl, 0)
        for t in range(4):
            pltpu.sync_copy(stage[0], acc.at[pl.ds(s * RPT + t * CH, CH)])

        @pl.when(s < NS - 1)
        def _():
            pltpu.sync_copy(stage[0], acc.at[pl.ds(s * RPT + 448, CH)])
            pltpu.sync_copy(stage[0].at[pl.ds(0, 72)],
                            acc.at[pl.ds(s * RPT + 560, 72)])

        @pl.when(s == NS - 1)
        def _():
            pltpu.sync_copy(stage[0].at[pl.ds(0, 88)],
                            acc.at[pl.ds(s * RPT + 448, 88)])
        plsc.subcore_barrier()

        for k in range(NSL):
            istart(k, k)
        iwait(0)
        gstart(0)
        iwait(1)
        gstart(1)

        def body(k, off):
            for b in range(NSL):
                j = k * NSL + b
                nb = (b + 2) % NSL

                @pl.when(j + 2 < CPT)
                def _():
                    iwait(nb)
                    gstart(nb)

                gwait(b)
                pltpu.sync_copy(stage[b], acc.at[didx[b]], add=True)

                for g in range(CH // 16):
                    d = kbuf[b][pl.ds(g * 16, 16)]
                    m = d > 0
                    plsc.store_compressed(dls.at[pl.ds(off, 16)],
                                          sidx[b][pl.ds(g * 16, 16)], mask=m)
                    plsc.store_compressed(dld.at[pl.ds(off, 16)],
                                          didx[b][pl.ds(g * 16, 16)], mask=m)
                    off = off + jnp.sum(d)

                @pl.when(j + NSL < CPT)
                def _():
                    istart(j + NSL, b)
            return off
        lax.fori_loop(0, CPT // NSL, body, jnp.int32(0))
        plsc.subcore_barrier()

        pltpu.sync_copy(acc.at[pl.ds(s * ORPT, ORPT)],
                        hpos.at[pl.ds(s * ORPT, ORPT)])

        @pl.when(s == NS - 1)
        def _():
            pltpu.sync_copy(acc.at[pl.ds(ORPT * NS, OREM)],
                            hpos.at[pl.ds(ORPT * NS, OREM)])
        plsc.subcore_barrier()

        def dcp(i, _):
            for ic in range(DCH):
                dld2[ic, pl.ds(i * 16, 16)] = dld[pl.ds(ic * CH + i * 16, 16)]
            return 0
        lax.fori_loop(0, CH // 16, dcp, 0)
        for ic in range(DCH):
            pltpu.async_copy(feat.at[dls.at[pl.ds(ic * CH, CH)]], stage[0],
                             gsem[0]).wait()
            pltpu.sync_copy(stage[0], acc.at[dld2.at[ic]], add=True)
        plsc.subcore_barrier()

        pltpu.sync_copy(acc.at[pl.ds(s * ORPT, ORPT)],
                        dump.at[pl.ds(s * ORPT, ORPT)])

        @pl.when(s == NS - 1)
        def _():
            pltpu.sync_copy(acc.at[pl.ds(ORPT * NS, OREM)],
                            dump.at[pl.ds(ORPT * NS, OREM)])

    @pl.when(c == 0)
    def _():
        run(feat_u, s_ub, d_ub, drop_ub, hpos_ub, dump_ub)

    @pl.when(c == 1)
    def _():
        run(feat_i, s_bu, d_bu, drop_bu, hpos_bu, dump_bu)


def _loss_body(aub, dub, abu, dbu, out):
    def one(a, dmp):
        b = 2.0 * a - dmp
        num = jnp.sum(a * b, axis=1)
        na = jnp.sqrt(jnp.sum(a * a, axis=1))
        nb = jnp.sqrt(jnp.sum(b * b, axis=1))
        cos = num / (jnp.maximum(na, 1e-8) * jnp.maximum(nb, 1e-8))
        return jnp.log(jnp.sum(jnp.exp(cos / TEM)))

    out[0, 0] = one(aub[...], dub[...]) + one(abu[...], dbu[...])


_tc_loss = pl.pallas_call(
    _loss_body,
    out_shape=jax.ShapeDtypeStruct((1, 1), jnp.float32),
    out_specs=pl.BlockSpec(memory_space=pltpu.SMEM),
)


def kernel(feat_items, feat_users, edges_ub_src, edges_ub_dst,
           edges_bu_src, edges_bu_dst):
    i32 = jnp.int32
    su = edges_ub_src.astype(i32)
    du = edges_ub_dst.astype(i32)
    sb = edges_bu_src.astype(i32)
    db = edges_bu_dst.astype(i32)

    pad = EPAD - E
    zpad = jnp.zeros((pad,), i32)
    spad = jnp.full((pad,), SENT, i32)
    su_p = jnp.concatenate([su, zpad])
    du_p = jnp.concatenate([du, spad])
    sb_p = jnp.concatenate([sb, zpad])
    db_p = jnp.concatenate([db, spad])

    drop_key = jax.random.key(42)
    k_ub, k_bu = jax.random.split(drop_key)
    keep_ub = jax.random.bernoulli(k_ub, p=1.0 - DROP, shape=(E,))
    keep_bu = jax.random.bernoulli(k_bu, p=1.0 - DROP, shape=(E,))
    dz = jnp.zeros((pad,), i32)
    drop_ub = jnp.concatenate([(~keep_ub).astype(i32), dz])
    drop_bu = jnp.concatenate([(~keep_bu).astype(i32), dz])

    hpos_ub, hpos_bu, dump_ub, dump_bu = _sc_msgpass(
        feat_users, feat_items, su_p, du_p, sb_p, db_p,
        drop_ub, drop_bu)

    loss = _tc_loss(hpos_ub, dump_ub, hpos_bu, dump_bu)[0, 0]
    return hpos_ub, hpos_bu, loss
